# Initial kernel scaffold; baseline (speedup 1.0000x reference)
import jax
import jax.numpy as jnp
from jax.experimental import pallas as pl


def kernel(region_scores_label, affinity_socres_label, region_scores_pre, affinity_scores_pre, mask):
    raise NotImplementedError("write your pallas kernel here")


# speedup vs baseline: 12.8715x; 12.8715x over previous
import jax
import jax.numpy as jnp
from jax import lax
from jax.experimental import pallas as pl
from jax.experimental.pallas import tpu as pltpu
from jax.experimental.pallas import tpu_sc as plsc

NC, NS, L = 2, 16, 16
NW = NC * NS
TOTAL = 8 * 384 * 384
PER_W = TOTAL // NW
CH = 4096
NCHUNK = PER_W // CH
NBIN = 1024
HISTW = L * NBIN

_mesh = plsc.VectorSubcoreMesh(
    core_axis_name="c", subcore_axis_name="s", num_cores=NC, num_subcores=NS)


def _zero_hists(h_cnt, h_sum, zeros):
    def zb(j, _):
        h_cnt[pl.ds(j * L, L)] = zeros
        h_sum[pl.ds(j * L, L)] = zeros
        return 0
    lax.fori_loop(0, HISTW // L, zb, 0)


def _reduce_hist(h, red, zeros):
    def rb(c, _):
        acc = zeros
        for l in range(L):
            acc = acc + h[pl.ds(l * NBIN + c * L, L)]
        red[pl.ds(c * L, L)] = acc
        return 0
    lax.fori_loop(0, NBIN // L, rb, 0)


def _phase1_body(rl, al, rp, ap, mm, v_r, v_a, hists, accs,
                 st_lab, st_pre, st_msk, st_v, h_cnt, h_sum, red, accv):
    wid = lax.axis_index("s") * NC + lax.axis_index("c")
    base = wid * PER_W
    lane_base = lax.iota(jnp.int32, L) * NBIN
    zeros = jnp.zeros((L,), jnp.float32)
    ones = jnp.ones((L,), jnp.float32)

    def do_tensor(lab_hbm, pre_hbm, v_hbm, cnt_row, sum_row, accp_row, accs_row):
        _zero_hists(h_cnt, h_sum, zeros)

        def chunk(c, carry):
            off = base + c * CH
            pltpu.sync_copy(lab_hbm.at[pl.ds(off, CH)], st_lab)
            pltpu.sync_copy(pre_hbm.at[pl.ds(off, CH)], st_pre)
            pltpu.sync_copy(mm.at[pl.ds(off, CH)], st_msk)

            def vec(i, carry2):
                aP, aS = carry2
                s = pl.ds(i * L, L)
                labv = st_lab[s]
                prev = st_pre[s]
                mskv = st_msk[s]
                dd = prev - labv
                pls_ = dd * dd * mskv
                pos = labv > 0.1
                aP = aP + jnp.where(pos, ones, zeros)
                aS = aS + jnp.where(pos, pls_, zeros)
                vv = jnp.where(pos, zeros, pls_)
                st_v[s] = vv
                bits = plsc.bitcast(vv, jnp.int32)
                addr = lane_base + (bits >> 21)
                plsc.addupdate_scatter(h_sum, [addr], vv)
                plsc.addupdate_scatter(h_cnt, [addr], ones)
                return aP, aS

            carry = lax.fori_loop(0, CH // L, vec, carry)
            pltpu.sync_copy(st_v, v_hbm.at[pl.ds(off, CH)])
            return carry

        accP, accS = lax.fori_loop(0, NCHUNK, chunk, (zeros, zeros))
        accv[accp_row] = accP
        accv[accs_row] = accS
        _reduce_hist(h_cnt, red, zeros)
        pltpu.sync_copy(red, hists.at[wid, cnt_row])
        _reduce_hist(h_sum, red, zeros)
        pltpu.sync_copy(red, hists.at[wid, sum_row])

    do_tensor(rl, rp, v_r, 0, 1, 0, 1)
    do_tensor(al, ap, v_a, 2, 3, 2, 3)
    pltpu.sync_copy(accv, accs.at[wid])


_phase1 = pl.kernel(
    _phase1_body,
    out_type=[
        jax.ShapeDtypeStruct((TOTAL,), jnp.float32),
        jax.ShapeDtypeStruct((TOTAL,), jnp.float32),
        jax.ShapeDtypeStruct((NW, 4, NBIN), jnp.float32),
        jax.ShapeDtypeStruct((NW, 4, L), jnp.float32),
    ],
    mesh=_mesh,
    compiler_params=pltpu.CompilerParams(needs_layout_passes=False),
    scratch_types=[
        pltpu.VMEM((CH,), jnp.float32),
        pltpu.VMEM((CH,), jnp.float32),
        pltpu.VMEM((CH,), jnp.float32),
        pltpu.VMEM((CH,), jnp.float32),
        pltpu.VMEM((HISTW,), jnp.float32),
        pltpu.VMEM((HISTW,), jnp.float32),
        pltpu.VMEM((NBIN,), jnp.float32),
        pltpu.VMEM((4, L), jnp.float32),
    ],
)


def _phase2_body(v_r, v_a, pref, hists, st_v, pref_v, h_cnt, h_sum, red):
    wid = lax.axis_index("s") * NC + lax.axis_index("c")
    base = wid * PER_W
    lane_base = lax.iota(jnp.int32, L) * NBIN
    zeros = jnp.zeros((L,), jnp.float32)
    ones = jnp.ones((L,), jnp.float32)
    pltpu.sync_copy(pref, pref_v)

    def do_tensor(v_hbm, prow, cnt_row, sum_row):
        _zero_hists(h_cnt, h_sum, zeros)
        b1v = pref_v[prow]

        def chunk(c, _):
            off = base + c * CH
            pltpu.sync_copy(v_hbm.at[pl.ds(off, CH)], st_v)

            def vec(i, __):
                vv = st_v[pl.ds(i * L, L)]
                bits = plsc.bitcast(vv, jnp.int32)
                match = (bits >> 21) == b1v
                addr = lane_base + ((bits >> 11) & 0x3FF)
                plsc.addupdate_scatter(h_sum, [addr], vv, mask=match)
                plsc.addupdate_scatter(h_cnt, [addr], ones, mask=match)
                return 0

            lax.fori_loop(0, CH // L, vec, 0)
            return 0

        lax.fori_loop(0, NCHUNK, chunk, 0)
        _reduce_hist(h_cnt, red, zeros)
        pltpu.sync_copy(red, hists.at[wid, cnt_row])
        _reduce_hist(h_sum, red, zeros)
        pltpu.sync_copy(red, hists.at[wid, sum_row])

    do_tensor(v_r, 0, 0, 1)
    do_tensor(v_a, 1, 2, 3)


_phase2 = pl.kernel(
    _phase2_body,
    out_type=jax.ShapeDtypeStruct((NW, 4, NBIN), jnp.float32),
    mesh=_mesh,
    compiler_params=pltpu.CompilerParams(needs_layout_passes=False),
    scratch_types=[
        pltpu.VMEM((CH,), jnp.float32),
        pltpu.VMEM((2, L), jnp.int32),
        pltpu.VMEM((HISTW,), jnp.float32),
        pltpu.VMEM((HISTW,), jnp.float32),
        pltpu.VMEM((NBIN,), jnp.float32),
    ],
)


def _select(cnt, s, k):
    cc = jnp.cumsum(cnt)
    cs = jnp.cumsum(s)
    above_c = cc[-1] - cc
    above_s = cs[-1] - cs
    hit = above_c < k
    b = jnp.where(jnp.any(hit), jnp.argmax(hit), 0).astype(jnp.int32)
    return b, k - above_c[b], above_s[b]


def kernel(region_scores_label, affinity_socres_label, region_scores_pre,
           affinity_scores_pre, mask):
    rl = region_scores_label.reshape(-1)
    al = affinity_socres_label.reshape(-1)
    rp = region_scores_pre.reshape(-1)
    ap = affinity_scores_pre.reshape(-1)
    mm = mask.reshape(-1)

    v_r, v_a, h1, acc = _phase1(rl, al, rp, ap, mm)

    cnt_r = jnp.sum(h1[:, 0], axis=0)
    sum_r = jnp.sum(h1[:, 1], axis=0)
    cnt_a = jnp.sum(h1[:, 2], axis=0)
    sum_a = jnp.sum(h1[:, 3], axis=0)
    P_r = jnp.sum(acc[:, 0])
    possum_r = jnp.sum(acc[:, 1])
    P_a = jnp.sum(acc[:, 2])
    possum_a = jnp.sum(acc[:, 3])
    negsum_r = jnp.sum(sum_r)
    negsum_a = jnp.sum(sum_a)

    k_r = jnp.floor(3.0 * P_r)
    k_a = jnp.floor(3.0 * P_a)
    b1r, k2r, above1_r = _select(cnt_r, sum_r, k_r)
    b1a, k2a, above1_a = _select(cnt_a, sum_a, k_a)

    pref = jnp.stack([jnp.full((L,), b1r, jnp.int32),
                      jnp.full((L,), b1a, jnp.int32)])
    h2 = _phase2(v_r, v_a, pref)

    b2r, kremr, above2_r = _select(jnp.sum(h2[:, 0], axis=0),
                                   jnp.sum(h2[:, 1], axis=0), k2r)
    b2a, krema, above2_a = _select(jnp.sum(h2[:, 2], axis=0),
                                   jnp.sum(h2[:, 3], axis=0), k2a)
    t_r = lax.bitcast_convert_type((b1r << 21) | (b2r << 11), jnp.float32)
    t_a = lax.bitcast_convert_type((b1a << 21) | (b2a << 11), jnp.float32)
    topk_r = above1_r + above2_r + kremr * t_r
    topk_a = above1_a + above2_a + krema * t_a

    total = jnp.float32(TOTAL)
    N_r = total - P_r
    N_a = total - P_a
    loss_r = possum_r / P_r + jnp.where(
        N_r < 3.0 * P_r, negsum_r / N_r, topk_r / (P_r * 3.0))
    loss_a = possum_a / P_a + jnp.where(
        N_a < 3.0 * P_a, negsum_a / N_a, topk_a / (P_a * 3.0))
    return loss_r + loss_a


# speedup vs baseline: 15.5368x; 1.2071x over previous
import jax
import jax.numpy as jnp
from jax import lax
from jax.experimental import pallas as pl
from jax.experimental.pallas import tpu as pltpu
from jax.experimental.pallas import tpu_sc as plsc

NC, NS, L = 2, 16, 16
NW = NC * NS
TOTAL = 8 * 384 * 384
PER_W = TOTAL // NW
CH = 9216
NCHUNK = PER_W // CH
NBIN = 1024
HISTW = L * NBIN

_mesh = plsc.VectorSubcoreMesh(
    core_axis_name="c", subcore_axis_name="s", num_cores=NC, num_subcores=NS)
_params = pltpu.CompilerParams(needs_layout_passes=False)


def _zero_hists(h_cnt, h_sum, zeros):
    def zb(j, _):
        h_cnt[pl.ds(j * L, L)] = zeros
        h_sum[pl.ds(j * L, L)] = zeros
        return 0
    lax.fori_loop(0, HISTW // L, zb, 0)


def _reduce_hist(h, red, zeros):
    def rb(c, _):
        acc = zeros
        for l in range(L):
            acc = acc + h[pl.ds(l * NBIN + c * L, L)]
        red[pl.ds(c * L, L)] = acc
        return 0
    lax.fori_loop(0, NBIN // L, rb, 0)


def _phase1_body(rl, al, rp, ap, mm, v_r, v_a, hists, accs,
                 st_lab0, st_lab1, st_pre0, st_pre1, st_msk0, st_msk1,
                 st_v0, st_v1, h_cnt, h_sum, red, accv,
                 semi0, semi1, semo0, semo1):
    wid = lax.axis_index("s") * NC + lax.axis_index("c")
    base = wid * PER_W
    lane_base = lax.iota(jnp.int32, L) * NBIN
    zeros = jnp.zeros((L,), jnp.float32)
    ones = jnp.ones((L,), jnp.float32)
    semi = (semi0, semi1)
    semo = (semo0, semo1)
    st_lab = (st_lab0, st_lab1)
    st_pre = (st_pre0, st_pre1)
    st_msk = (st_msk0, st_msk1)
    st_v = (st_v0, st_v1)

    def do_tensor(lab_hbm, pre_hbm, v_hbm, cnt_row, sum_row, accp_row, accs_row):
        _zero_hists(h_cnt, h_sum, zeros)
        in_h = {}
        out_h = {}

        def issue_in(c):
            buf = c & 1
            off = base + c * CH
            in_h[c] = [
                pltpu.async_copy(lab_hbm.at[pl.ds(off, CH)], st_lab[buf], semi[buf]),
                pltpu.async_copy(pre_hbm.at[pl.ds(off, CH)], st_pre[buf], semi[buf]),
                pltpu.async_copy(mm.at[pl.ds(off, CH)], st_msk[buf], semi[buf]),
            ]

        issue_in(0)
        accP, accS = zeros, zeros
        for c in range(NCHUNK):
            buf = c & 1
            if c + 1 < NCHUNK:
                issue_in(c + 1)
            for h in in_h.pop(c):
                h.wait()
            if c >= 2:
                out_h.pop(c - 2).wait()
            labr = st_lab[buf]
            prer = st_pre[buf]
            mskr = st_msk[buf]
            vr = st_v[buf]

            def vec(i, carry2):
                aP, aS = carry2
                s = pl.ds(i * L, L)
                labv = labr[s]
                prev = prer[s]
                mskv = mskr[s]
                dd = prev - labv
                pls_ = dd * dd * mskv
                pos = labv > 0.1
                aP = aP + jnp.where(pos, ones, zeros)
                aS = aS + jnp.where(pos, pls_, zeros)
                vv = jnp.where(pos, zeros, pls_)
                vr[s] = vv
                bits = plsc.bitcast(vv, jnp.int32)
                addr = lane_base + (bits >> 21)
                plsc.addupdate_scatter(h_sum, [addr], vv)
                plsc.addupdate_scatter(h_cnt, [addr], ones)
                return aP, aS

            accP, accS = lax.fori_loop(0, CH // L, vec, (accP, accS))
            out_h[c] = pltpu.async_copy(
                vr, v_hbm.at[pl.ds(base + c * CH, CH)], semo[buf])
        for c in sorted(out_h):
            out_h.pop(c).wait()

        accv[accp_row] = accP
        accv[accs_row] = accS
        _reduce_hist(h_cnt, red, zeros)
        pltpu.sync_copy(red, hists.at[wid, cnt_row])
        _reduce_hist(h_sum, red, zeros)
        pltpu.sync_copy(red, hists.at[wid, sum_row])

    do_tensor(rl, rp, v_r, 0, 1, 0, 1)
    do_tensor(al, ap, v_a, 2, 3, 2, 3)
    pltpu.sync_copy(accv, accs.at[wid])


_phase1 = pl.kernel(
    _phase1_body,
    out_type=[
        jax.ShapeDtypeStruct((TOTAL,), jnp.float32),
        jax.ShapeDtypeStruct((TOTAL,), jnp.float32),
        jax.ShapeDtypeStruct((NW, 4, NBIN), jnp.float32),
        jax.ShapeDtypeStruct((NW, 4, L), jnp.float32),
    ],
    mesh=_mesh,
    compiler_params=_params,
    scratch_types=[
        pltpu.VMEM((CH,), jnp.float32),
        pltpu.VMEM((CH,), jnp.float32),
        pltpu.VMEM((CH,), jnp.float32),
        pltpu.VMEM((CH,), jnp.float32),
        pltpu.VMEM((CH,), jnp.float32),
        pltpu.VMEM((CH,), jnp.float32),
        pltpu.VMEM((CH,), jnp.float32),
        pltpu.VMEM((CH,), jnp.float32),
        pltpu.VMEM((HISTW,), jnp.float32),
        pltpu.VMEM((HISTW,), jnp.float32),
        pltpu.VMEM((NBIN,), jnp.float32),
        pltpu.VMEM((4, L), jnp.float32),
        pltpu.SemaphoreType.DMA,
        pltpu.SemaphoreType.DMA,
        pltpu.SemaphoreType.DMA,
        pltpu.SemaphoreType.DMA,
    ],
)


def _phase2_body(v_r, v_a, pref, hists, st_v0, st_v1, pref_v, h_cnt, h_sum, red,
                 semi0, semi1):
    wid = lax.axis_index("s") * NC + lax.axis_index("c")
    base = wid * PER_W
    lane_base = lax.iota(jnp.int32, L) * NBIN
    zeros = jnp.zeros((L,), jnp.float32)
    ones = jnp.ones((L,), jnp.float32)
    semi = (semi0, semi1)
    st_v = (st_v0, st_v1)
    pltpu.sync_copy(pref, pref_v)

    def do_tensor(v_hbm, prow, cnt_row, sum_row):
        _zero_hists(h_cnt, h_sum, zeros)
        b1v = pref_v[prow]
        in_h = {}

        def issue_in(c):
            buf = c & 1
            in_h[c] = pltpu.async_copy(
                v_hbm.at[pl.ds(base + c * CH, CH)], st_v[buf], semi[buf])

        issue_in(0)
        for c in range(NCHUNK):
            buf = c & 1
            if c + 1 < NCHUNK:
                issue_in(c + 1)
            in_h.pop(c).wait()
            vvr = st_v[buf]

            def vec(i, _):
                vv = vvr[pl.ds(i * L, L)]
                bits = plsc.bitcast(vv, jnp.int32)
                match = (bits >> 21) == b1v
                addr = lane_base + ((bits >> 11) & 0x3FF)
                plsc.addupdate_scatter(h_sum, [addr], vv, mask=match)
                plsc.addupdate_scatter(h_cnt, [addr], ones, mask=match)
                return 0

            lax.fori_loop(0, CH // L, vec, 0)
        _reduce_hist(h_cnt, red, zeros)
        pltpu.sync_copy(red, hists.at[wid, cnt_row])
        _reduce_hist(h_sum, red, zeros)
        pltpu.sync_copy(red, hists.at[wid, sum_row])

    do_tensor(v_r, 0, 0, 1)
    do_tensor(v_a, 1, 2, 3)


_phase2 = pl.kernel(
    _phase2_body,
    out_type=jax.ShapeDtypeStruct((NW, 4, NBIN), jnp.float32),
    mesh=_mesh,
    compiler_params=_params,
    scratch_types=[
        pltpu.VMEM((CH,), jnp.float32),
        pltpu.VMEM((CH,), jnp.float32),
        pltpu.VMEM((2, L), jnp.int32),
        pltpu.VMEM((HISTW,), jnp.float32),
        pltpu.VMEM((HISTW,), jnp.float32),
        pltpu.VMEM((NBIN,), jnp.float32),
        pltpu.SemaphoreType.DMA,
        pltpu.SemaphoreType.DMA,
    ],
)


def _select(cnt, s, k):
    cc = jnp.cumsum(cnt)
    cs = jnp.cumsum(s)
    above_c = cc[-1] - cc
    above_s = cs[-1] - cs
    hit = above_c < k
    b = jnp.where(jnp.any(hit), jnp.argmax(hit), 0).astype(jnp.int32)
    return b, k - above_c[b], above_s[b]


def kernel(region_scores_label, affinity_socres_label, region_scores_pre,
           affinity_scores_pre, mask):
    rl = region_scores_label.reshape(-1)
    al = affinity_socres_label.reshape(-1)
    rp = region_scores_pre.reshape(-1)
    ap = affinity_scores_pre.reshape(-1)
    mm = mask.reshape(-1)

    v_r, v_a, h1, acc = _phase1(rl, al, rp, ap, mm)

    cnt_r = jnp.sum(h1[:, 0], axis=0)
    sum_r = jnp.sum(h1[:, 1], axis=0)
    cnt_a = jnp.sum(h1[:, 2], axis=0)
    sum_a = jnp.sum(h1[:, 3], axis=0)
    P_r = jnp.sum(acc[:, 0])
    possum_r = jnp.sum(acc[:, 1])
    P_a = jnp.sum(acc[:, 2])
    possum_a = jnp.sum(acc[:, 3])
    negsum_r = jnp.sum(sum_r)
    negsum_a = jnp.sum(sum_a)

    k_r = jnp.floor(3.0 * P_r)
    k_a = jnp.floor(3.0 * P_a)
    b1r, k2r, above1_r = _select(cnt_r, sum_r, k_r)
    b1a, k2a, above1_a = _select(cnt_a, sum_a, k_a)

    pref = jnp.stack([jnp.full((L,), b1r, jnp.int32),
                      jnp.full((L,), b1a, jnp.int32)])
    h2 = _phase2(v_r, v_a, pref)

    b2r, kremr, above2_r = _select(jnp.sum(h2[:, 0], axis=0),
                                   jnp.sum(h2[:, 1], axis=0), k2r)
    b2a, krema, above2_a = _select(jnp.sum(h2[:, 2], axis=0),
                                   jnp.sum(h2[:, 3], axis=0), k2a)
    t_r = lax.bitcast_convert_type((b1r << 21) | (b2r << 11), jnp.float32)
    t_a = lax.bitcast_convert_type((b1a << 21) | (b2a << 11), jnp.float32)
    topk_r = above1_r + above2_r + kremr * t_r
    topk_a = above1_a + above2_a + krema * t_a

    total = jnp.float32(TOTAL)
    N_r = total - P_r
    N_a = total - P_a
    loss_r = possum_r / P_r + jnp.where(
        N_r < 3.0 * P_r, negsum_r / N_r, topk_r / (P_r * 3.0))
    loss_a = possum_a / P_a + jnp.where(
        N_a < 3.0 * P_a, negsum_a / N_a, topk_a / (P_a * 3.0))
    return loss_r + loss_a


# speedup vs baseline: 16.4737x; 1.0603x over previous
import jax
import jax.numpy as jnp
from jax import lax
from jax.experimental import pallas as pl
from jax.experimental.pallas import tpu as pltpu
from jax.experimental.pallas import tpu_sc as plsc

NC, NS, L = 2, 16, 16
NW = NC * NS
TOTAL = 8 * 384 * 384
PER_W = TOTAL // NW
CH = 9216
NCHUNK = PER_W // CH
NBIN = 1024
HISTW = L * NBIN

_mesh = plsc.VectorSubcoreMesh(
    core_axis_name="c", subcore_axis_name="s", num_cores=NC, num_subcores=NS)
_params = pltpu.CompilerParams(needs_layout_passes=False)


def _zero_hists(h_cnt, h_sum, zeros):
    def zb(j, _):
        for u in range(8):
            h_cnt[pl.ds(j * 8 * L + u * L, L)] = zeros
            h_sum[pl.ds(j * 8 * L + u * L, L)] = zeros
        return 0
    lax.fori_loop(0, HISTW // (8 * L), zb, 0)


def _reduce_hist(h, red, zeros):
    def rb(c, _):
        acc = zeros
        for l in range(L):
            acc = acc + h[pl.ds(l * NBIN + c * L, L)]
        red[pl.ds(c * L, L)] = acc
        return 0
    lax.fori_loop(0, NBIN // L, rb, 0)


def _phase1_body(rl, al, rp, ap, mm, v_r, v_a, hists, accs,
                 st_lab0, st_lab1, st_pre0, st_pre1, st_msk0, st_msk1,
                 st_v0, st_v1, h_cnt, h_sum, red, accv,
                 semi0, semi1, semo0, semo1):
    wid = lax.axis_index("s") * NC + lax.axis_index("c")
    base = wid * PER_W
    lane_base = lax.iota(jnp.int32, L) * NBIN
    zeros = jnp.zeros((L,), jnp.float32)
    ones = jnp.ones((L,), jnp.float32)
    semi = (semi0, semi1)
    semo = (semo0, semo1)
    st_lab = (st_lab0, st_lab1)
    st_pre = (st_pre0, st_pre1)
    st_msk = (st_msk0, st_msk1)
    st_v = (st_v0, st_v1)

    def do_tensor(lab_hbm, pre_hbm, v_hbm, cnt_row, sum_row, accp_row, accs_row):
        _zero_hists(h_cnt, h_sum, zeros)
        in_h = {}
        out_h = {}

        def issue_in(c):
            buf = c & 1
            off = base + c * CH
            in_h[c] = [
                pltpu.async_copy(lab_hbm.at[pl.ds(off, CH)], st_lab[buf], semi[buf]),
                pltpu.async_copy(pre_hbm.at[pl.ds(off, CH)], st_pre[buf], semi[buf]),
                pltpu.async_copy(mm.at[pl.ds(off, CH)], st_msk[buf], semi[buf]),
            ]

        issue_in(0)
        accP, accS = zeros, zeros
        for c in range(NCHUNK):
            buf = c & 1
            if c + 1 < NCHUNK:
                issue_in(c + 1)
            for h in in_h.pop(c):
                h.wait()
            if c >= 2:
                out_h.pop(c - 2).wait()
            labr = st_lab[buf]
            prer = st_pre[buf]
            mskr = st_msk[buf]
            vr = st_v[buf]

            def vec(i, carry2):
                aP, aS = carry2
                for u in range(4):
                    s = pl.ds(i * 4 * L + u * L, L)
                    labv = labr[s]
                    prev = prer[s]
                    mskv = mskr[s]
                    dd = prev - labv
                    pls_ = dd * dd * mskv
                    pos = labv > 0.1
                    aP = aP + jnp.where(pos, ones, zeros)
                    aS = aS + jnp.where(pos, pls_, zeros)
                    vv = jnp.where(pos, zeros, pls_)
                    vr[s] = vv
                    bits = plsc.bitcast(vv, jnp.int32)
                    addr = lane_base + (bits >> 21)
                    plsc.addupdate_scatter(h_sum, [addr], vv)
                    plsc.addupdate_scatter(h_cnt, [addr], ones)
                return aP, aS

            accP, accS = lax.fori_loop(0, CH // (4 * L), vec, (accP, accS))
            out_h[c] = pltpu.async_copy(
                vr, v_hbm.at[pl.ds(base + c * CH, CH)], semo[buf])
        for c in sorted(out_h):
            out_h.pop(c).wait()

        accv[accp_row] = accP
        accv[accs_row] = accS
        _reduce_hist(h_cnt, red, zeros)
        pltpu.sync_copy(red, hists.at[wid, cnt_row])
        _reduce_hist(h_sum, red, zeros)
        pltpu.sync_copy(red, hists.at[wid, sum_row])

    do_tensor(rl, rp, v_r, 0, 1, 0, 1)
    do_tensor(al, ap, v_a, 2, 3, 2, 3)
    pltpu.sync_copy(accv, accs.at[wid])


_phase1 = pl.kernel(
    _phase1_body,
    out_type=[
        jax.ShapeDtypeStruct((TOTAL,), jnp.float32),
        jax.ShapeDtypeStruct((TOTAL,), jnp.float32),
        jax.ShapeDtypeStruct((NW, 4, NBIN), jnp.float32),
        jax.ShapeDtypeStruct((NW, 4, L), jnp.float32),
    ],
    mesh=_mesh,
    compiler_params=_params,
    scratch_types=[
        pltpu.VMEM((CH,), jnp.float32),
        pltpu.VMEM((CH,), jnp.float32),
        pltpu.VMEM((CH,), jnp.float32),
        pltpu.VMEM((CH,), jnp.float32),
        pltpu.VMEM((CH,), jnp.float32),
        pltpu.VMEM((CH,), jnp.float32),
        pltpu.VMEM((CH,), jnp.float32),
        pltpu.VMEM((CH,), jnp.float32),
        pltpu.VMEM((HISTW,), jnp.float32),
        pltpu.VMEM((HISTW,), jnp.float32),
        pltpu.VMEM((NBIN,), jnp.float32),
        pltpu.VMEM((4, L), jnp.float32),
        pltpu.SemaphoreType.DMA,
        pltpu.SemaphoreType.DMA,
        pltpu.SemaphoreType.DMA,
        pltpu.SemaphoreType.DMA,
    ],
)


def _phase2_body(v_r, v_a, pref, hists, st_v0, st_v1, pref_v, h_cnt, h_sum, red,
                 semi0, semi1):
    wid = lax.axis_index("s") * NC + lax.axis_index("c")
    base = wid * PER_W
    lane_base = lax.iota(jnp.int32, L) * NBIN
    zeros = jnp.zeros((L,), jnp.float32)
    ones = jnp.ones((L,), jnp.float32)
    semi = (semi0, semi1)
    st_v = (st_v0, st_v1)
    pltpu.sync_copy(pref, pref_v)

    def do_tensor(v_hbm, prow, cnt_row, sum_row):
        _zero_hists(h_cnt, h_sum, zeros)
        b1v = pref_v[prow]
        in_h = {}

        def issue_in(c):
            buf = c & 1
            in_h[c] = pltpu.async_copy(
                v_hbm.at[pl.ds(base + c * CH, CH)], st_v[buf], semi[buf])

        issue_in(0)
        for c in range(NCHUNK):
            buf = c & 1
            if c + 1 < NCHUNK:
                issue_in(c + 1)
            in_h.pop(c).wait()
            vvr = st_v[buf]

            def vec(i, _):
                for u in range(4):
                    vv = vvr[pl.ds(i * 4 * L + u * L, L)]
                    bits = plsc.bitcast(vv, jnp.int32)
                    match = (bits >> 21) == b1v
                    addr = lane_base + ((bits >> 11) & 0x3FF)
                    plsc.addupdate_scatter(h_sum, [addr], vv, mask=match)
                    plsc.addupdate_scatter(h_cnt, [addr], ones, mask=match)
                return 0

            lax.fori_loop(0, CH // (4 * L), vec, 0)
        _reduce_hist(h_cnt, red, zeros)
        pltpu.sync_copy(red, hists.at[wid, cnt_row])
        _reduce_hist(h_sum, red, zeros)
        pltpu.sync_copy(red, hists.at[wid, sum_row])

    do_tensor(v_r, 0, 0, 1)
    do_tensor(v_a, 1, 2, 3)


_phase2 = pl.kernel(
    _phase2_body,
    out_type=jax.ShapeDtypeStruct((NW, 4, NBIN), jnp.float32),
    mesh=_mesh,
    compiler_params=_params,
    scratch_types=[
        pltpu.VMEM((CH,), jnp.float32),
        pltpu.VMEM((CH,), jnp.float32),
        pltpu.VMEM((2, L), jnp.int32),
        pltpu.VMEM((HISTW,), jnp.float32),
        pltpu.VMEM((HISTW,), jnp.float32),
        pltpu.VMEM((NBIN,), jnp.float32),
        pltpu.SemaphoreType.DMA,
        pltpu.SemaphoreType.DMA,
    ],
)


def _select(cnt, s, k):
    cc = jnp.cumsum(cnt)
    cs = jnp.cumsum(s)
    above_c = cc[-1] - cc
    above_s = cs[-1] - cs
    hit = above_c < k
    b = jnp.where(jnp.any(hit), jnp.argmax(hit), 0).astype(jnp.int32)
    return b, k - above_c[b], above_s[b]


def kernel(region_scores_label, affinity_socres_label, region_scores_pre,
           affinity_scores_pre, mask):
    rl = region_scores_label.reshape(-1)
    al = affinity_socres_label.reshape(-1)
    rp = region_scores_pre.reshape(-1)
    ap = affinity_scores_pre.reshape(-1)
    mm = mask.reshape(-1)

    v_r, v_a, h1, acc = _phase1(rl, al, rp, ap, mm)

    cnt_r = jnp.sum(h1[:, 0], axis=0)
    sum_r = jnp.sum(h1[:, 1], axis=0)
    cnt_a = jnp.sum(h1[:, 2], axis=0)
    sum_a = jnp.sum(h1[:, 3], axis=0)
    P_r = jnp.sum(acc[:, 0])
    possum_r = jnp.sum(acc[:, 1])
    P_a = jnp.sum(acc[:, 2])
    possum_a = jnp.sum(acc[:, 3])
    negsum_r = jnp.sum(sum_r)
    negsum_a = jnp.sum(sum_a)

    k_r = jnp.floor(3.0 * P_r)
    k_a = jnp.floor(3.0 * P_a)
    b1r, k2r, above1_r = _select(cnt_r, sum_r, k_r)
    b1a, k2a, above1_a = _select(cnt_a, sum_a, k_a)

    pref = jnp.stack([jnp.full((L,), b1r, jnp.int32),
                      jnp.full((L,), b1a, jnp.int32)])
    h2 = _phase2(v_r, v_a, pref)

    b2r, kremr, above2_r = _select(jnp.sum(h2[:, 0], axis=0),
                                   jnp.sum(h2[:, 1], axis=0), k2r)
    b2a, krema, above2_a = _select(jnp.sum(h2[:, 2], axis=0),
                                   jnp.sum(h2[:, 3], axis=0), k2a)
    t_r = lax.bitcast_convert_type((b1r << 21) | (b2r << 11), jnp.float32)
    t_a = lax.bitcast_convert_type((b1a << 21) | (b2a << 11), jnp.float32)
    topk_r = above1_r + above2_r + kremr * t_r
    topk_a = above1_a + above2_a + krema * t_a

    total = jnp.float32(TOTAL)
    N_r = total - P_r
    N_a = total - P_a
    loss_r = possum_r / P_r + jnp.where(
        N_r < 3.0 * P_r, negsum_r / N_r, topk_r / (P_r * 3.0))
    loss_a = possum_a / P_a + jnp.where(
        N_a < 3.0 * P_a, negsum_a / N_a, topk_a / (P_a * 3.0))
    return loss_r + loss_a


# speedup vs baseline: 23.4817x; 1.4254x over previous
import jax
import jax.numpy as jnp
from jax import lax
from jax.experimental import pallas as pl
from jax.experimental.pallas import tpu as pltpu
from jax.experimental.pallas import tpu_sc as plsc

NC, NS, L = 2, 16, 16
NW = NC * NS
TOTAL = 8 * 384 * 384
PER_W = TOTAL // NW
CH = 9216
NCHUNK = PER_W // CH
NBIN = 1024
NBINH = 1040
HISTW = L * NBINH
U = 4

_mesh = plsc.VectorSubcoreMesh(
    core_axis_name="c", subcore_axis_name="s", num_cores=NC, num_subcores=NS)
_params = pltpu.CompilerParams(needs_layout_passes=False)


def _zero_hists(h_cnt, h_sum, zeros):
    def zb(j, _):
        for u in range(8):
            h_cnt[pl.ds(j * 8 * L + u * L, L)] = zeros
            h_sum[pl.ds(j * 8 * L + u * L, L)] = zeros
        return 0
    lax.fori_loop(0, HISTW // (8 * L), zb, 0)


def _reduce_hist(h, red, zeros):
    def rb(c, _):
        acc = zeros
        for l in range(L):
            acc = acc + h[pl.ds(l * NBINH + c * L, L)]
        red[pl.ds(c * L, L)] = acc
        return 0
    lax.fori_loop(0, NBINH // L, rb, 0)


def _phase1_body(rl, al, rp, ap, mm, v_r, v_a, hists,
                 st_lab0, st_lab1, st_pre0, st_pre1, st_msk0, st_msk1,
                 st_v0, st_v1, h_cnt, h_sum, red,
                 semi0, semi1, semo0, semo1):
    wid = lax.axis_index("s") * NC + lax.axis_index("c")
    base = wid * PER_W
    lane_base = lax.iota(jnp.int32, L) * NBINH
    zeros = jnp.zeros((L,), jnp.float32)
    ones = jnp.ones((L,), jnp.float32)
    posbin = jnp.full((L,), NBIN, jnp.int32)
    semi = (semi0, semi1)
    semo = (semo0, semo1)
    st_lab = (st_lab0, st_lab1)
    st_pre = (st_pre0, st_pre1)
    st_msk = (st_msk0, st_msk1)
    st_v = (st_v0, st_v1)

    def do_tensor(lab_hbm, pre_hbm, v_hbm, cnt_row, sum_row):
        _zero_hists(h_cnt, h_sum, zeros)
        in_h = {}
        out_h = {}

        def issue_in(c):
            buf = c & 1
            off = base + c * CH
            in_h[c] = [
                pltpu.async_copy(lab_hbm.at[pl.ds(off, CH)], st_lab[buf], semi[buf]),
                pltpu.async_copy(pre_hbm.at[pl.ds(off, CH)], st_pre[buf], semi[buf]),
                pltpu.async_copy(mm.at[pl.ds(off, CH)], st_msk[buf], semi[buf]),
            ]

        issue_in(0)
        for c in range(NCHUNK):
            buf = c & 1
            if c + 1 < NCHUNK:
                issue_in(c + 1)
            for h in in_h.pop(c):
                h.wait()
            if c >= 2:
                out_h.pop(c - 2).wait()
            labr = st_lab[buf]
            prer = st_pre[buf]
            mskr = st_msk[buf]
            vr = st_v[buf]

            def vec(i, _):
                ss = [pl.ds(i * U * L + u * L, L) for u in range(U)]
                labs = [labr[s] for s in ss]
                pres = [prer[s] for s in ss]
                msks = [mskr[s] for s in ss]
                dds = [pres[u] - labs[u] for u in range(U)]
                sqs = [dds[u] * dds[u] for u in range(U)]
                plss = [sqs[u] * msks[u] for u in range(U)]
                poss = [labs[u] > 0.1 for u in range(U)]
                vvs = [jnp.where(poss[u], zeros, plss[u]) for u in range(U)]
                bits = [plsc.bitcast(vvs[u], jnp.int32) >> 21 for u in range(U)]
                addrs = [lane_base + jnp.where(poss[u], posbin, bits[u])
                         for u in range(U)]
                for u in range(U):
                    vr[ss[u]] = vvs[u]
                for u in range(U):
                    plsc.addupdate_scatter(h_sum, [addrs[u]], plss[u])
                    plsc.addupdate_scatter(h_cnt, [addrs[u]], ones)
                return 0

            lax.fori_loop(0, CH // (U * L), vec, 0)
            out_h[c] = pltpu.async_copy(
                vr, v_hbm.at[pl.ds(base + c * CH, CH)], semo[buf])
        for c in sorted(out_h):
            out_h.pop(c).wait()

        _reduce_hist(h_cnt, red, zeros)
        pltpu.sync_copy(red, hists.at[wid, cnt_row])
        _reduce_hist(h_sum, red, zeros)
        pltpu.sync_copy(red, hists.at[wid, sum_row])

    do_tensor(rl, rp, v_r, 0, 1)
    do_tensor(al, ap, v_a, 2, 3)


_phase1 = pl.kernel(
    _phase1_body,
    out_type=[
        jax.ShapeDtypeStruct((TOTAL,), jnp.float32),
        jax.ShapeDtypeStruct((TOTAL,), jnp.float32),
        jax.ShapeDtypeStruct((NW, 4, NBINH), jnp.float32),
    ],
    mesh=_mesh,
    compiler_params=_params,
    scratch_types=[
        pltpu.VMEM((CH,), jnp.float32),
        pltpu.VMEM((CH,), jnp.float32),
        pltpu.VMEM((CH,), jnp.float32),
        pltpu.VMEM((CH,), jnp.float32),
        pltpu.VMEM((CH,), jnp.float32),
        pltpu.VMEM((CH,), jnp.float32),
        pltpu.VMEM((CH,), jnp.float32),
        pltpu.VMEM((CH,), jnp.float32),
        pltpu.VMEM((HISTW,), jnp.float32),
        pltpu.VMEM((HISTW,), jnp.float32),
        pltpu.VMEM((NBINH,), jnp.float32),
        pltpu.SemaphoreType.DMA,
        pltpu.SemaphoreType.DMA,
        pltpu.SemaphoreType.DMA,
        pltpu.SemaphoreType.DMA,
    ],
)


def _phase2_body(v_r, v_a, pref, hists, st_v0, st_v1, pref_v, h_cnt, h_sum, red,
                 semi0, semi1):
    wid = lax.axis_index("s") * NC + lax.axis_index("c")
    base = wid * PER_W
    lane_base = lax.iota(jnp.int32, L) * NBINH
    zeros = jnp.zeros((L,), jnp.float32)
    ones = jnp.ones((L,), jnp.float32)
    semi = (semi0, semi1)
    st_v = (st_v0, st_v1)
    pltpu.sync_copy(pref, pref_v)

    def do_tensor(v_hbm, prow, cnt_row, sum_row):
        _zero_hists(h_cnt, h_sum, zeros)
        b1v = pref_v[prow]
        in_h = {}

        def issue_in(c):
            buf = c & 1
            in_h[c] = pltpu.async_copy(
                v_hbm.at[pl.ds(base + c * CH, CH)], st_v[buf], semi[buf])

        issue_in(0)
        for c in range(NCHUNK):
            buf = c & 1
            if c + 1 < NCHUNK:
                issue_in(c + 1)
            in_h.pop(c).wait()
            vvr = st_v[buf]

            def vec(i, _):
                ss = [pl.ds(i * U * L + u * L, L) for u in range(U)]
                vvs = [vvr[s] for s in ss]
                bits = [plsc.bitcast(vvs[u], jnp.int32) for u in range(U)]
                matches = [(bits[u] >> 21) == b1v for u in range(U)]
                addrs = [lane_base + ((bits[u] >> 11) & 0x3FF) for u in range(U)]
                for u in range(U):
                    plsc.addupdate_scatter(h_sum, [addrs[u]], vvs[u],
                                           mask=matches[u])
                    plsc.addupdate_scatter(h_cnt, [addrs[u]], ones,
                                           mask=matches[u])
                return 0

            lax.fori_loop(0, CH // (U * L), vec, 0)
        _reduce_hist(h_cnt, red, zeros)
        pltpu.sync_copy(red, hists.at[wid, cnt_row])
        _reduce_hist(h_sum, red, zeros)
        pltpu.sync_copy(red, hists.at[wid, sum_row])

    do_tensor(v_r, 0, 0, 1)
    do_tensor(v_a, 1, 2, 3)


_phase2 = pl.kernel(
    _phase2_body,
    out_type=jax.ShapeDtypeStruct((NW, 4, NBINH), jnp.float32),
    mesh=_mesh,
    compiler_params=_params,
    scratch_types=[
        pltpu.VMEM((CH,), jnp.float32),
        pltpu.VMEM((CH,), jnp.float32),
        pltpu.VMEM((2, L), jnp.int32),
        pltpu.VMEM((HISTW,), jnp.float32),
        pltpu.VMEM((HISTW,), jnp.float32),
        pltpu.VMEM((NBINH,), jnp.float32),
        pltpu.SemaphoreType.DMA,
        pltpu.SemaphoreType.DMA,
    ],
)


def _select(cnt, s, k):
    cc = jnp.cumsum(cnt)
    cs = jnp.cumsum(s)
    above_c = cc[-1] - cc
    above_s = cs[-1] - cs
    hit = above_c < k
    b = jnp.where(jnp.any(hit), jnp.argmax(hit), 0).astype(jnp.int32)
    return b, k - above_c[b], above_s[b]


def kernel(region_scores_label, affinity_socres_label, region_scores_pre,
           affinity_scores_pre, mask):
    rl = region_scores_label.reshape(-1)
    al = affinity_socres_label.reshape(-1)
    rp = region_scores_pre.reshape(-1)
    ap = affinity_scores_pre.reshape(-1)
    mm = mask.reshape(-1)

    v_r, v_a, h1 = _phase1(rl, al, rp, ap, mm)

    cnt_r = jnp.sum(h1[:, 0, :NBIN], axis=0)
    sum_r = jnp.sum(h1[:, 1, :NBIN], axis=0)
    cnt_a = jnp.sum(h1[:, 2, :NBIN], axis=0)
    sum_a = jnp.sum(h1[:, 3, :NBIN], axis=0)
    P_r = jnp.sum(h1[:, 0, NBIN])
    possum_r = jnp.sum(h1[:, 1, NBIN])
    P_a = jnp.sum(h1[:, 2, NBIN])
    possum_a = jnp.sum(h1[:, 3, NBIN])
    negsum_r = jnp.sum(sum_r)
    negsum_a = jnp.sum(sum_a)

    k_r = jnp.floor(3.0 * P_r)
    k_a = jnp.floor(3.0 * P_a)
    b1r, k2r, above1_r = _select(cnt_r, sum_r, k_r)
    b1a, k2a, above1_a = _select(cnt_a, sum_a, k_a)

    pref = jnp.stack([jnp.full((L,), b1r, jnp.int32),
                      jnp.full((L,), b1a, jnp.int32)])
    h2 = _phase2(v_r, v_a, pref)

    b2r, kremr, above2_r = _select(jnp.sum(h2[:, 0, :NBIN], axis=0),
                                   jnp.sum(h2[:, 1, :NBIN], axis=0), k2r)
    b2a, krema, above2_a = _select(jnp.sum(h2[:, 2, :NBIN], axis=0),
                                   jnp.sum(h2[:, 3, :NBIN], axis=0), k2a)
    t_r = lax.bitcast_convert_type((b1r << 21) | (b2r << 11), jnp.float32)
    t_a = lax.bitcast_convert_type((b1a << 21) | (b2a << 11), jnp.float32)
    topk_r = above1_r + above2_r + kremr * t_r
    topk_a = above1_a + above2_a + krema * t_a

    total = jnp.float32(TOTAL)
    N_r = total - P_r
    N_a = total - P_a
    loss_r = possum_r / P_r + jnp.where(
        N_r < 3.0 * P_r, negsum_r / N_r, topk_r / (P_r * 3.0))
    loss_a = possum_a / P_a + jnp.where(
        N_a < 3.0 * P_a, negsum_a / N_a, topk_a / (P_a * 3.0))
    return loss_r + loss_a


# speedup vs baseline: 24.6769x; 1.0509x over previous
import jax
import jax.numpy as jnp
from jax import lax
from jax.experimental import pallas as pl
from jax.experimental.pallas import tpu as pltpu
from jax.experimental.pallas import tpu_sc as plsc

NC, NS, L = 2, 16, 16
NW = NC * NS
TOTAL = 8 * 384 * 384
PER_W = TOTAL // NW
CH = 9216
NCHUNK = PER_W // CH
NBIN = 1024
NBINH = 1040
HISTW = L * NBINH
U = 8

_mesh = plsc.VectorSubcoreMesh(
    core_axis_name="c", subcore_axis_name="s", num_cores=NC, num_subcores=NS)
_params = pltpu.CompilerParams(needs_layout_passes=False)


def _zero_hists(h_cnt, h_sum, zeros):
    def zb(j, _):
        for u in range(8):
            h_cnt[pl.ds(j * 8 * L + u * L, L)] = zeros
            h_sum[pl.ds(j * 8 * L + u * L, L)] = zeros
        return 0
    lax.fori_loop(0, HISTW // (8 * L), zb, 0)


def _reduce_hist(h, red, zeros):
    def rb(c, _):
        acc = zeros
        for l in range(L):
            acc = acc + h[pl.ds(l * NBINH + c * L, L)]
        red[pl.ds(c * L, L)] = acc
        return 0
    lax.fori_loop(0, NBINH // L, rb, 0)


def _phase1_body(rl, al, rp, ap, mm, v_r, v_a, hists,
                 st_lab0, st_lab1, st_pre0, st_pre1, st_msk0, st_msk1,
                 st_v0, st_v1, h_cnt, h_sum, red,
                 semi0, semi1, semo0, semo1):
    wid = lax.axis_index("s") * NC + lax.axis_index("c")
    base = wid * PER_W
    lane_base = lax.iota(jnp.int32, L) * NBINH
    zeros = jnp.zeros((L,), jnp.float32)
    ones = jnp.ones((L,), jnp.float32)
    posbin = jnp.full((L,), NBIN, jnp.int32)
    semi = (semi0, semi1)
    semo = (semo0, semo1)
    st_lab = (st_lab0, st_lab1)
    st_pre = (st_pre0, st_pre1)
    st_msk = (st_msk0, st_msk1)
    st_v = (st_v0, st_v1)

    def do_tensor(lab_hbm, pre_hbm, v_hbm, cnt_row, sum_row):
        _zero_hists(h_cnt, h_sum, zeros)
        in_h = {}
        out_h = {}

        def issue_in(c):
            buf = c & 1
            off = base + c * CH
            in_h[c] = [
                pltpu.async_copy(lab_hbm.at[pl.ds(off, CH)], st_lab[buf], semi[buf]),
                pltpu.async_copy(pre_hbm.at[pl.ds(off, CH)], st_pre[buf], semi[buf]),
                pltpu.async_copy(mm.at[pl.ds(off, CH)], st_msk[buf], semi[buf]),
            ]

        issue_in(0)
        for c in range(NCHUNK):
            buf = c & 1
            if c + 1 < NCHUNK:
                issue_in(c + 1)
            for h in in_h.pop(c):
                h.wait()
            if c >= 2:
                out_h.pop(c - 2).wait()
            labr = st_lab[buf]
            prer = st_pre[buf]
            mskr = st_msk[buf]
            vr = st_v[buf]

            def vec(i, _):
                ss = [pl.ds(i * U * L + u * L, L) for u in range(U)]
                labs = [labr[s] for s in ss]
                pres = [prer[s] for s in ss]
                msks = [mskr[s] for s in ss]
                dds = [pres[u] - labs[u] for u in range(U)]
                sqs = [dds[u] * dds[u] for u in range(U)]
                plss = [sqs[u] * msks[u] for u in range(U)]
                poss = [labs[u] > 0.1 for u in range(U)]
                vvs = [jnp.where(poss[u], zeros, plss[u]) for u in range(U)]
                bits = [plsc.bitcast(vvs[u], jnp.int32) >> 21 for u in range(U)]
                addrs = [lane_base + jnp.where(poss[u], posbin, bits[u])
                         for u in range(U)]
                for u in range(U):
                    vr[ss[u]] = vvs[u]
                for u in range(U):
                    plsc.addupdate_scatter(h_sum, [addrs[u]], plss[u])
                    plsc.addupdate_scatter(h_cnt, [addrs[u]], ones)
                return 0

            lax.fori_loop(0, CH // (U * L), vec, 0)
            out_h[c] = pltpu.async_copy(
                vr, v_hbm.at[pl.ds(base + c * CH, CH)], semo[buf])
        for c in sorted(out_h):
            out_h.pop(c).wait()

        _reduce_hist(h_cnt, red, zeros)
        pltpu.sync_copy(red, hists.at[wid, cnt_row])
        _reduce_hist(h_sum, red, zeros)
        pltpu.sync_copy(red, hists.at[wid, sum_row])

    do_tensor(rl, rp, v_r, 0, 1)
    do_tensor(al, ap, v_a, 2, 3)


_phase1 = pl.kernel(
    _phase1_body,
    out_type=[
        jax.ShapeDtypeStruct((TOTAL,), jnp.float32),
        jax.ShapeDtypeStruct((TOTAL,), jnp.float32),
        jax.ShapeDtypeStruct((NW, 4, NBINH), jnp.float32),
    ],
    mesh=_mesh,
    compiler_params=_params,
    scratch_types=[
        pltpu.VMEM((CH,), jnp.float32),
        pltpu.VMEM((CH,), jnp.float32),
        pltpu.VMEM((CH,), jnp.float32),
        pltpu.VMEM((CH,), jnp.float32),
        pltpu.VMEM((CH,), jnp.float32),
        pltpu.VMEM((CH,), jnp.float32),
        pltpu.VMEM((CH,), jnp.float32),
        pltpu.VMEM((CH,), jnp.float32),
        pltpu.VMEM((HISTW,), jnp.float32),
        pltpu.VMEM((HISTW,), jnp.float32),
        pltpu.VMEM((NBINH,), jnp.float32),
        pltpu.SemaphoreType.DMA,
        pltpu.SemaphoreType.DMA,
        pltpu.SemaphoreType.DMA,
        pltpu.SemaphoreType.DMA,
    ],
)


def _phase2_body(v_r, v_a, pref, hists, st_v0, st_v1, pref_v, h_cnt, h_sum, red,
                 semi0, semi1):
    wid = lax.axis_index("s") * NC + lax.axis_index("c")
    base = wid * PER_W
    lane_base = lax.iota(jnp.int32, L) * NBINH
    zeros = jnp.zeros((L,), jnp.float32)
    ones = jnp.ones((L,), jnp.float32)
    semi = (semi0, semi1)
    st_v = (st_v0, st_v1)
    pltpu.sync_copy(pref, pref_v)

    def do_tensor(v_hbm, prow, cnt_row, sum_row):
        _zero_hists(h_cnt, h_sum, zeros)
        b1v = pref_v[prow]
        in_h = {}

        def issue_in(c):
            buf = c & 1
            in_h[c] = pltpu.async_copy(
                v_hbm.at[pl.ds(base + c * CH, CH)], st_v[buf], semi[buf])

        issue_in(0)
        for c in range(NCHUNK):
            buf = c & 1
            if c + 1 < NCHUNK:
                issue_in(c + 1)
            in_h.pop(c).wait()
            vvr = st_v[buf]

            def vec(i, _):
                ss = [pl.ds(i * U * L + u * L, L) for u in range(U)]
                vvs = [vvr[s] for s in ss]
                bits = [plsc.bitcast(vvs[u], jnp.int32) for u in range(U)]
                matches = [(bits[u] >> 21) == b1v for u in range(U)]
                addrs = [lane_base + ((bits[u] >> 11) & 0x3FF) for u in range(U)]
                for u in range(U):
                    plsc.addupdate_scatter(h_sum, [addrs[u]], vvs[u],
                                           mask=matches[u])
                    plsc.addupdate_scatter(h_cnt, [addrs[u]], ones,
                                           mask=matches[u])
                return 0

            lax.fori_loop(0, CH // (U * L), vec, 0)
        _reduce_hist(h_cnt, red, zeros)
        pltpu.sync_copy(red, hists.at[wid, cnt_row])
        _reduce_hist(h_sum, red, zeros)
        pltpu.sync_copy(red, hists.at[wid, sum_row])

    do_tensor(v_r, 0, 0, 1)
    do_tensor(v_a, 1, 2, 3)


_phase2 = pl.kernel(
    _phase2_body,
    out_type=jax.ShapeDtypeStruct((NW, 4, NBINH), jnp.float32),
    mesh=_mesh,
    compiler_params=_params,
    scratch_types=[
        pltpu.VMEM((CH,), jnp.float32),
        pltpu.VMEM((CH,), jnp.float32),
        pltpu.VMEM((2, L), jnp.int32),
        pltpu.VMEM((HISTW,), jnp.float32),
        pltpu.VMEM((HISTW,), jnp.float32),
        pltpu.VMEM((NBINH,), jnp.float32),
        pltpu.SemaphoreType.DMA,
        pltpu.SemaphoreType.DMA,
    ],
)


def _select(cnt, s, k):
    cc = jnp.cumsum(cnt)
    cs = jnp.cumsum(s)
    above_c = cc[-1] - cc
    above_s = cs[-1] - cs
    hit = above_c < k
    b = jnp.where(jnp.any(hit), jnp.argmax(hit), 0).astype(jnp.int32)
    return b, k - above_c[b], above_s[b]


def kernel(region_scores_label, affinity_socres_label, region_scores_pre,
           affinity_scores_pre, mask):
    rl = region_scores_label.reshape(-1)
    al = affinity_socres_label.reshape(-1)
    rp = region_scores_pre.reshape(-1)
    ap = affinity_scores_pre.reshape(-1)
    mm = mask.reshape(-1)

    v_r, v_a, h1 = _phase1(rl, al, rp, ap, mm)

    cnt_r = jnp.sum(h1[:, 0, :NBIN], axis=0)
    sum_r = jnp.sum(h1[:, 1, :NBIN], axis=0)
    cnt_a = jnp.sum(h1[:, 2, :NBIN], axis=0)
    sum_a = jnp.sum(h1[:, 3, :NBIN], axis=0)
    P_r = jnp.sum(h1[:, 0, NBIN])
    possum_r = jnp.sum(h1[:, 1, NBIN])
    P_a = jnp.sum(h1[:, 2, NBIN])
    possum_a = jnp.sum(h1[:, 3, NBIN])
    negsum_r = jnp.sum(sum_r)
    negsum_a = jnp.sum(sum_a)

    k_r = jnp.floor(3.0 * P_r)
    k_a = jnp.floor(3.0 * P_a)
    b1r, k2r, above1_r = _select(cnt_r, sum_r, k_r)
    b1a, k2a, above1_a = _select(cnt_a, sum_a, k_a)

    pref = jnp.stack([jnp.full((L,), b1r, jnp.int32),
                      jnp.full((L,), b1a, jnp.int32)])
    h2 = _phase2(v_r, v_a, pref)

    b2r, kremr, above2_r = _select(jnp.sum(h2[:, 0, :NBIN], axis=0),
                                   jnp.sum(h2[:, 1, :NBIN], axis=0), k2r)
    b2a, krema, above2_a = _select(jnp.sum(h2[:, 2, :NBIN], axis=0),
                                   jnp.sum(h2[:, 3, :NBIN], axis=0), k2a)
    t_r = lax.bitcast_convert_type((b1r << 21) | (b2r << 11), jnp.float32)
    t_a = lax.bitcast_convert_type((b1a << 21) | (b2a << 11), jnp.float32)
    topk_r = above1_r + above2_r + kremr * t_r
    topk_a = above1_a + above2_a + krema * t_a

    total = jnp.float32(TOTAL)
    N_r = total - P_r
    N_a = total - P_a
    loss_r = possum_r / P_r + jnp.where(
        N_r < 3.0 * P_r, negsum_r / N_r, topk_r / (P_r * 3.0))
    loss_a = possum_a / P_a + jnp.where(
        N_a < 3.0 * P_a, negsum_a / N_a, topk_a / (P_a * 3.0))
    return loss_r + loss_a
